# baseline (speedup 1.0000x reference)
import jax
import jax.numpy as jnp
from jax import lax
from jax.experimental import pallas as pl
from jax.experimental.pallas import tpu as pltpu
from jax.experimental.pallas import tpu_sc as plsc

N = 50000
C = 128
K = 27
E = 20000
BN_EPS = 1e-5

NC, NS, L = 2, 16, 16
NW = NC * NS

E_PAD = 20480
KE_PAD = K * E_PAD
IDX_ROWS = KE_PAD // 128
EW = KE_PAD // NW

GR = 128
G_CHUNKS = EW // GR
SR = 24
S_CHUNKS = IDX_ROWS // SR

N_ACC = 51200
DUMMY_DST = N

MM_TILE = 2048
RED_TILE = 2048


PW = E // (NW - 1) // 32 * 32
TAILW = E_PAD - (NW - 1) * PW


def _pad_body(sf_hbm, df_hbm, so_hbm, do_hbm, sbuf, dbuf, s2, d2, sem):
    c = lax.axis_index("c")
    s = lax.axis_index("s")
    wid = s * NC + c

    @pl.when(wid < NW - 1)
    def _():
        w0 = wid * PW
        sb, db = (sbuf, s2), (dbuf, d2)

        def wait_w(kk):
            b = kk % 2
            pltpu.make_async_copy(sb[b], so_hbm.at[pl.ds(0, PW)],
                                  sem).wait()
            pltpu.make_async_copy(db[b], do_hbm.at[pl.ds(0, PW)],
                                  sem).wait()

        for k in range(K):
            b = k % 2
            if k >= 2:
                wait_w(k - 2)
            pltpu.sync_copy(sf_hbm.at[pl.ds(k * E + w0, PW)], sb[b])
            pltpu.sync_copy(df_hbm.at[pl.ds(k * E + w0, PW)], db[b])
            pltpu.async_copy(sb[b], so_hbm.at[pl.ds(k * E_PAD + w0, PW)],
                             sem)
            pltpu.async_copy(db[b], do_hbm.at[pl.ds(k * E_PAD + w0, PW)],
                             sem)
        wait_w(K - 2)
        wait_w(K - 1)

    @pl.when(wid == NW - 1)
    def _():
        tail = E - (NW - 1) * PW
        def fill(i, carry):
            sbuf[pl.ds(tail + i * L, L)] = jnp.zeros((L,), jnp.int32)
            dbuf[pl.ds(tail + i * L, L)] = jnp.full((L,), DUMMY_DST,
                                                    jnp.int32)
            return carry
        lax.fori_loop(0, (TAILW - tail) // L, fill, 0)
        for k in range(K):
            pltpu.sync_copy(sf_hbm.at[pl.ds(k * E + (NW - 1) * PW, tail)],
                            sbuf.at[pl.ds(0, tail)])
            pltpu.sync_copy(df_hbm.at[pl.ds(k * E + (NW - 1) * PW, tail)],
                            dbuf.at[pl.ds(0, tail)])
            pltpu.sync_copy(sbuf,
                            so_hbm.at[pl.ds(k * E_PAD + (NW - 1) * PW,
                                            TAILW)])
            pltpu.sync_copy(dbuf,
                            do_hbm.at[pl.ds(k * E_PAD + (NW - 1) * PW,
                                            TAILW)])


def _sc_pad(src_flat, dst_flat):
    mesh = plsc.VectorSubcoreMesh(core_axis_name="c", subcore_axis_name="s")
    return pl.kernel(
        _pad_body,
        out_type=(jax.ShapeDtypeStruct((KE_PAD,), jnp.int32),
                  jax.ShapeDtypeStruct((KE_PAD,), jnp.int32)),
        mesh=mesh,
        scratch_types=[
            pltpu.VMEM((TAILW,), jnp.int32),
            pltpu.VMEM((TAILW,), jnp.int32),
            pltpu.VMEM((TAILW,), jnp.int32),
            pltpu.VMEM((TAILW,), jnp.int32),
            pltpu.SemaphoreType.DMA,
        ],
        compiler_params=pltpu.CompilerParams(needs_layout_passes=False),
    )(src_flat, dst_flat)


NBUF = 5


def _gather_body(feats_hbm, src_hbm, g_hbm, idx_v, *bufs_and_sems):
    bufs = bufs_and_sems[:NBUF]
    gsems = bufs_and_sems[NBUF:2 * NBUF]
    wsems = bufs_and_sems[2 * NBUF:3 * NBUF]
    c = lax.axis_index("c")
    s = lax.axis_index("s")
    wid = s * NC + c
    row0 = wid * (EW // 128)
    pltpu.sync_copy(src_hbm.at[wid], idx_v)

    def fire_g(j, b):
        pltpu.async_copy(feats_hbm.at[idx_v.at[j]], bufs[b], gsems[b])

    def wait_g(b):
        pltpu.make_async_copy(feats_hbm.at[pl.ds(0, GR)],
                              bufs[b], gsems[b]).wait()

    def fire_w(j, b):
        off = pl.multiple_of((row0 + j) * GR, GR)
        pltpu.async_copy(bufs[b], g_hbm.at[pl.ds(off, GR)], wsems[b])

    def wait_w(b):
        pltpu.make_async_copy(bufs[b], g_hbm.at[pl.ds(0, GR)],
                              wsems[b]).wait()

    def body(i, carry):
        for b in range(NBUF):
            @pl.when(i > 0)
            def _():
                wait_w(b)
            fire_g(i * NBUF + b, b)
        for b in range(NBUF):
            wait_g(b)
            fire_w(i * NBUF + b, b)
        return carry

    lax.fori_loop(0, G_CHUNKS // NBUF, body, 0)
    for b in range(NBUF):
        wait_w(b)


def _sc_gather(feats, src3):
    mesh = plsc.VectorSubcoreMesh(core_axis_name="c", subcore_axis_name="s")
    return pl.kernel(
        _gather_body,
        out_type=jax.ShapeDtypeStruct((KE_PAD, C), jnp.float32),
        mesh=mesh,
        scratch_types=[pltpu.VMEM((EW // 128, 128), jnp.int32)]
        + [pltpu.VMEM((GR, C), jnp.float32)] * NBUF
        + [pltpu.SemaphoreType.DMA] * (2 * NBUF),
        compiler_params=pltpu.CompilerParams(needs_layout_passes=False),
    )(feats, src3)


SBUF = 3


def _scatter_body(mt_hbm, dst_hbm, ot_hbm, acc0, acc1, *bufs_and_sems):
    dbufs = bufs_and_sems[:SBUF]
    mbufs = bufs_and_sems[SBUF:2 * SBUF]
    sems = bufs_and_sems[2 * SBUF:3 * SBUF]
    c = lax.axis_index("c")
    s = lax.axis_index("s")
    wid = s * NC + c

    for p in range(2):
        rp = p * NW + wid

        def zero_row(r, carry):
            z = jnp.zeros((L,), jnp.float32)
            acc0[pl.ds(r * L, L)] = z
            acc1[pl.ds(r * L, L)] = z
            return carry

        lax.fori_loop(0, N_ACC // L, zero_row, 0)

        def fire(j, b):
            off = pl.multiple_of(j * SR, SR)
            pltpu.async_copy(dst_hbm.at[pl.ds(off, SR)], dbufs[b], sems[b])
            pltpu.async_copy(mt_hbm.at[rp, pl.ds(off, SR)], mbufs[b], sems[b])

        def wait(b):
            pltpu.make_async_copy(dst_hbm.at[pl.ds(0, SR)],
                                  dbufs[b], sems[b]).wait()
            pltpu.make_async_copy(mt_hbm.at[0, pl.ds(0, SR)],
                                  mbufs[b], sems[b]).wait()

        def compute(b):
            dbuf, mbuf = dbufs[b], mbufs[b]

            def inner(r, carry2):
                for v in range(8):
                    d = dbuf[r, pl.ds(v * L, L)]
                    m = mbuf[r, pl.ds(v * L, L)]
                    fe = plsc.bitcast(lax.shift_left(m, 16), jnp.float32)
                    fo = plsc.bitcast(
                        jnp.bitwise_and(m, jnp.int32(-65536)), jnp.float32)
                    plsc.addupdate_scatter(acc0, [d], fe)
                    plsc.addupdate_scatter(acc1, [d], fo)
                return carry2
            lax.fori_loop(0, SR, inner, 0)

        for b in range(SBUF):
            fire(b, b)

        def body(i, carry):
            for b in range(SBUF):
                wait(b)
                compute(b)

                @pl.when(i < S_CHUNKS // SBUF - 1)
                def _():
                    fire((i + 1) * SBUF + b, b)
            return carry

        lax.fori_loop(0, S_CHUNKS // SBUF, body, 0)
        pltpu.sync_copy(acc0, ot_hbm.at[pl.ds((2 * rp) * N_ACC, N_ACC)])
        pltpu.sync_copy(acc1, ot_hbm.at[pl.ds((2 * rp + 1) * N_ACC, N_ACC)])


def _sc_scatter(mt3, dst2):
    mesh = plsc.VectorSubcoreMesh(core_axis_name="c", subcore_axis_name="s")
    return pl.kernel(
        _scatter_body,
        out_type=jax.ShapeDtypeStruct((C * N_ACC,), jnp.float32),
        mesh=mesh,
        scratch_types=[
            pltpu.VMEM((N_ACC,), jnp.float32),
            pltpu.VMEM((N_ACC,), jnp.float32),
        ]
        + [pltpu.VMEM((SR, 128), jnp.int32)] * SBUF
        + [pltpu.VMEM((SR, 128), jnp.int32)] * SBUF
        + [pltpu.SemaphoreType.DMA] * SBUF,
        compiler_params=pltpu.CompilerParams(needs_layout_passes=False),
    )(mt3, dst2)


def _rte_bf16_bits(u):
    lsb = jnp.bitwise_and(lax.shift_right_logical(u, 16), 1)
    return lax.shift_right_logical(u + 0x7FFF + lsb, 16)


def _mm_body(g_ref, we_ref, wo_ref, o_ref):
    g = g_ref[...]
    dn = (((0,), (1,)), ((), ()))
    me = lax.dot_general(we_ref[0], g, dimension_numbers=dn,
                         preferred_element_type=jnp.float32)
    mo = lax.dot_general(wo_ref[0], g, dimension_numbers=dn,
                         preferred_element_type=jnp.float32)
    be = _rte_bf16_bits(lax.bitcast_convert_type(me, jnp.int32))
    bo = _rte_bf16_bits(lax.bitcast_convert_type(mo, jnp.int32))
    o_ref[...] = jnp.bitwise_or(lax.shift_left(bo, 16), be)


def _tc_matmul_t(g, we, wo):
    return pl.pallas_call(
        _mm_body,
        grid=(KE_PAD // MM_TILE,),
        in_specs=[
            pl.BlockSpec((MM_TILE, C), lambda i: (i, 0)),
            pl.BlockSpec((1, C, C // 2),
                         lambda i: (i // (E_PAD // MM_TILE), 0, 0)),
            pl.BlockSpec((1, C, C // 2),
                         lambda i: (i // (E_PAD // MM_TILE), 0, 0)),
        ],
        out_specs=pl.BlockSpec((C // 2, MM_TILE), lambda i: (0, i)),
        out_shape=jax.ShapeDtypeStruct((C // 2, KE_PAD), jnp.int32),
    )(g, we, wo)


def _red_body(x_ref, o_ref):
    i = pl.program_id(0)
    x = x_ref[...]
    col = lax.broadcasted_iota(jnp.int32, (C, RED_TILE), 1) + i * RED_TILE
    x = jnp.where(col < N, x, 0.0)
    ps = jnp.sum(x, axis=1, keepdims=True)
    pss = jnp.sum(x * x, axis=1, keepdims=True)

    @pl.when(i == 0)
    def _():
        o_ref[...] = jnp.zeros_like(o_ref)

    o_ref[:, 0:1] = o_ref[:, 0:1] + ps
    o_ref[:, 1:2] = o_ref[:, 1:2] + pss


def _tc_reduce(ot2):
    return pl.pallas_call(
        _red_body,
        grid=(N_ACC // RED_TILE,),
        in_specs=[pl.BlockSpec((C, RED_TILE), lambda i: (0, i))],
        out_specs=pl.BlockSpec((C, 128), lambda i: (0, 0)),
        out_shape=jax.ShapeDtypeStruct((C, 128), jnp.float32),
    )(ot2)


def _apply_body(x_ref, st_ref, gb_ref, o_ref):
    x = x_ref[...]
    inv_n = 1.0 / N
    mean = st_ref[:, 0:1] * inv_n
    var = st_ref[:, 1:2] * inv_n - mean * mean
    scale = gb_ref[:, 0:1] * lax.rsqrt(var + BN_EPS)
    shift = gb_ref[:, 1:2] - mean * scale
    y = jnp.maximum(x * scale + shift, 0.0)
    r = lax.broadcasted_iota(jnp.int32, (C, C), 0)
    cc = lax.broadcasted_iota(jnp.int32, (C, C), 1)
    eye = jnp.where(r == cc, 1.0, 0.0).astype(jnp.float32)
    o_ref[...] = lax.dot_general(
        y, eye, dimension_numbers=(((0,), (0,)), ((), ())),
        preferred_element_type=jnp.float32)


def _tc_apply(ot2, stats, gb):
    return pl.pallas_call(
        _apply_body,
        grid=(N_ACC // RED_TILE,),
        in_specs=[
            pl.BlockSpec((C, RED_TILE), lambda i: (0, i)),
            pl.BlockSpec((C, 128), lambda i: (0, 0)),
            pl.BlockSpec((C, 128), lambda i: (0, 0)),
        ],
        out_specs=pl.BlockSpec((RED_TILE, C), lambda i: (i, 0)),
        out_shape=jax.ShapeDtypeStruct((N, C), jnp.float32),
    )(ot2, stats, gb)


@jax.jit
def kernel(feats, edge_src, edge_dst, W, gamma, beta):
    src_p, dst_p = _sc_pad(edge_src.astype(jnp.int32).reshape(-1),
                           edge_dst.astype(jnp.int32).reshape(-1))
    src3 = src_p.reshape(NW, EW // 128, 128)
    dst2 = dst_p.reshape(IDX_ROWS, 128)

    g = _sc_gather(feats, src3)
    mt = _tc_matmul_t(g, W[:, :, 0::2], W[:, :, 1::2])
    ot = _sc_scatter(mt.reshape(C // 2, IDX_ROWS, 128), dst2)
    ot2 = ot.reshape(C, N_ACC)
    stats = _tc_reduce(ot2)
    gb = jnp.zeros((C, 128), jnp.float32)
    gb = gb.at[:, 0].set(gamma).at[:, 1].set(beta)
    return _tc_apply(ot2, stats, gb)


# speedup vs baseline: 1.0933x; 1.0933x over previous
import jax
import jax.numpy as jnp
from jax import lax
from jax.experimental import pallas as pl
from jax.experimental.pallas import tpu as pltpu
from jax.experimental.pallas import tpu_sc as plsc

N = 50000
C = 128
K = 27
E = 20000
BN_EPS = 1e-5

NC, NS, L = 2, 16, 16
NW = NC * NS

E_PAD = 20480
KE_PAD = K * E_PAD
IDX_ROWS = KE_PAD // 128
EW = KE_PAD // NW

GR = 128
G_CHUNKS = EW // GR
SR = 24
S_CHUNKS = IDX_ROWS // SR

N_ACC = 51200
DUMMY_DST = N

MM_TILE = 2048
RED_TILE = 2048


PW = E // (NW - 1) // 32 * 32
TAILW = E_PAD - (NW - 1) * PW


def _pad_body(sf_hbm, df_hbm, so_hbm, do_hbm, sbuf, dbuf, s2, d2, sem):
    c = lax.axis_index("c")
    s = lax.axis_index("s")
    wid = s * NC + c

    @pl.when(wid < NW - 1)
    def _():
        w0 = wid * PW
        sb, db = (sbuf, s2), (dbuf, d2)

        def wait_w(kk):
            b = kk % 2
            pltpu.make_async_copy(sb[b], so_hbm.at[pl.ds(0, PW)],
                                  sem).wait()
            pltpu.make_async_copy(db[b], do_hbm.at[pl.ds(0, PW)],
                                  sem).wait()

        for k in range(K):
            b = k % 2
            if k >= 2:
                wait_w(k - 2)
            pltpu.sync_copy(sf_hbm.at[pl.ds(k * E + w0, PW)], sb[b])
            pltpu.sync_copy(df_hbm.at[pl.ds(k * E + w0, PW)], db[b])
            pltpu.async_copy(sb[b], so_hbm.at[pl.ds(k * E_PAD + w0, PW)],
                             sem)
            pltpu.async_copy(db[b], do_hbm.at[pl.ds(k * E_PAD + w0, PW)],
                             sem)
        wait_w(K - 2)
        wait_w(K - 1)

    @pl.when(wid == NW - 1)
    def _():
        tail = E - (NW - 1) * PW
        def fill(i, carry):
            sbuf[pl.ds(tail + i * L, L)] = jnp.zeros((L,), jnp.int32)
            dbuf[pl.ds(tail + i * L, L)] = jnp.full((L,), DUMMY_DST,
                                                    jnp.int32)
            return carry
        lax.fori_loop(0, (TAILW - tail) // L, fill, 0)
        for k in range(K):
            pltpu.sync_copy(sf_hbm.at[pl.ds(k * E + (NW - 1) * PW, tail)],
                            sbuf.at[pl.ds(0, tail)])
            pltpu.sync_copy(df_hbm.at[pl.ds(k * E + (NW - 1) * PW, tail)],
                            dbuf.at[pl.ds(0, tail)])
            pltpu.sync_copy(sbuf,
                            so_hbm.at[pl.ds(k * E_PAD + (NW - 1) * PW,
                                            TAILW)])
            pltpu.sync_copy(dbuf,
                            do_hbm.at[pl.ds(k * E_PAD + (NW - 1) * PW,
                                            TAILW)])


def _sc_pad(src_flat, dst_flat):
    mesh = plsc.VectorSubcoreMesh(core_axis_name="c", subcore_axis_name="s")
    return pl.kernel(
        _pad_body,
        out_type=(jax.ShapeDtypeStruct((KE_PAD,), jnp.int32),
                  jax.ShapeDtypeStruct((KE_PAD,), jnp.int32)),
        mesh=mesh,
        scratch_types=[
            pltpu.VMEM((TAILW,), jnp.int32),
            pltpu.VMEM((TAILW,), jnp.int32),
            pltpu.VMEM((TAILW,), jnp.int32),
            pltpu.VMEM((TAILW,), jnp.int32),
            pltpu.SemaphoreType.DMA,
        ],
        compiler_params=pltpu.CompilerParams(needs_layout_passes=False),
    )(src_flat, dst_flat)


NBUF = 5
NSEG = 3
KSEG = K // NSEG
E_SEG = KSEG * E_PAD
CW_SEG = E_SEG // NW // 128


def _make_gather_body(seg):
    def _gather_body(feats_hbm, src_hbm, g_hbm, idx_v, *bufs_and_sems):
        bufs = bufs_and_sems[:NBUF]
        gsems = bufs_and_sems[NBUF:2 * NBUF]
        wsems = bufs_and_sems[2 * NBUF:3 * NBUF]
        c = lax.axis_index("c")
        s = lax.axis_index("s")
        wid = s * NC + c
        row0 = wid * CW_SEG
        pltpu.sync_copy(src_hbm.at[seg * NW + wid], idx_v)

        def fire_g(j, b):
            pltpu.async_copy(feats_hbm.at[idx_v.at[j]], bufs[b], gsems[b])

        def wait_g(b):
            pltpu.make_async_copy(feats_hbm.at[pl.ds(0, GR)],
                                  bufs[b], gsems[b]).wait()

        def fire_w(j, b):
            off = pl.multiple_of((row0 + j) * GR, GR)
            pltpu.async_copy(bufs[b], g_hbm.at[pl.ds(off, GR)], wsems[b])

        def wait_w(b):
            pltpu.make_async_copy(bufs[b], g_hbm.at[pl.ds(0, GR)],
                                  wsems[b]).wait()

        def body(i, carry):
            for b in range(NBUF):
                @pl.when(i > 0)
                def _():
                    wait_w(b)
                fire_g(i * NBUF + b, b)
            for b in range(NBUF):
                wait_g(b)
                fire_w(i * NBUF + b, b)
            return carry

        lax.fori_loop(0, CW_SEG // NBUF, body, 0)
        for b in range(NBUF):
            wait_w(b)
    return _gather_body


def _sc_gather(feats, src4, seg):
    mesh = plsc.VectorSubcoreMesh(core_axis_name="c", subcore_axis_name="s")
    return pl.kernel(
        _make_gather_body(seg),
        out_type=jax.ShapeDtypeStruct((E_SEG, C), jnp.float32),
        mesh=mesh,
        scratch_types=[pltpu.VMEM((CW_SEG, 128), jnp.int32)]
        + [pltpu.VMEM((GR, C), jnp.float32)] * NBUF
        + [pltpu.SemaphoreType.DMA] * (2 * NBUF),
        compiler_params=pltpu.CompilerParams(needs_layout_passes=False),
        name=f"gather_seg{seg}",
    )(feats, src4)


SBUF = 3


SC_SEG = (E_SEG // 128) // SR


def _scatter_body(mt0_hbm, mt1_hbm, mt2_hbm, dst_hbm, ot_hbm,
                  acc0, acc1, *bufs_and_sems):
    dbufs = bufs_and_sems[:SBUF]
    mbufs = bufs_and_sems[SBUF:2 * SBUF]
    sems = bufs_and_sems[2 * SBUF:3 * SBUF]
    c = lax.axis_index("c")
    s = lax.axis_index("s")
    wid = s * NC + c
    mts = (mt0_hbm, mt1_hbm, mt2_hbm)

    for p in range(2):
        rp = p * NW + wid

        def zero_row(r, carry):
            z = jnp.zeros((L,), jnp.float32)
            acc0[pl.ds(r * L, L)] = z
            acc1[pl.ds(r * L, L)] = z
            return carry

        lax.fori_loop(0, N_ACC // L, zero_row, 0)

        for seg in range(NSEG):
            mt_hbm = mts[seg]

            def fire(j, b):
                off = pl.multiple_of(j * SR, SR)
                pltpu.async_copy(
                    dst_hbm.at[pl.ds(seg * (E_SEG // 128) + j * SR, SR)],
                    dbufs[b], sems[b])
                pltpu.async_copy(mt_hbm.at[rp, pl.ds(off, SR)],
                                 mbufs[b], sems[b])

            def wait(b):
                pltpu.make_async_copy(dst_hbm.at[pl.ds(0, SR)],
                                      dbufs[b], sems[b]).wait()
                pltpu.make_async_copy(mt_hbm.at[0, pl.ds(0, SR)],
                                      mbufs[b], sems[b]).wait()

            def compute(b):
                dbuf, mbuf = dbufs[b], mbufs[b]

                def inner(r, carry2):
                    for v in range(8):
                        d = dbuf[r, pl.ds(v * L, L)]
                        m = mbuf[r, pl.ds(v * L, L)]
                        fe = plsc.bitcast(lax.shift_left(m, 16), jnp.float32)
                        fo = plsc.bitcast(
                            jnp.bitwise_and(m, jnp.int32(-65536)),
                            jnp.float32)
                        plsc.addupdate_scatter(acc0, [d], fe)
                        plsc.addupdate_scatter(acc1, [d], fo)
                    return carry2
                lax.fori_loop(0, SR, inner, 0)

            for b in range(SBUF):
                fire(b, b)

            def body(i, carry):
                for b in range(SBUF):
                    wait(b)
                    compute(b)

                    @pl.when(i < SC_SEG // SBUF - 1)
                    def _():
                        fire((i + 1) * SBUF + b, b)
                return carry

            lax.fori_loop(0, SC_SEG // SBUF, body, 0)
        pltpu.sync_copy(acc0, ot_hbm.at[pl.ds((2 * rp) * N_ACC, N_ACC)])
        pltpu.sync_copy(acc1, ot_hbm.at[pl.ds((2 * rp + 1) * N_ACC, N_ACC)])


def _sc_scatter(mt3s, dst2):
    mesh = plsc.VectorSubcoreMesh(core_axis_name="c", subcore_axis_name="s")
    return pl.kernel(
        _scatter_body,
        out_type=jax.ShapeDtypeStruct((C * N_ACC,), jnp.float32),
        mesh=mesh,
        scratch_types=[
            pltpu.VMEM((N_ACC,), jnp.float32),
            pltpu.VMEM((N_ACC,), jnp.float32),
        ]
        + [pltpu.VMEM((SR, 128), jnp.int32)] * SBUF
        + [pltpu.VMEM((SR, 128), jnp.int32)] * SBUF
        + [pltpu.SemaphoreType.DMA] * SBUF,
        compiler_params=pltpu.CompilerParams(needs_layout_passes=False),
    )(*mt3s, dst2)


def _rte_bf16_bits(u):
    lsb = jnp.bitwise_and(lax.shift_right_logical(u, 16), 1)
    return lax.shift_right_logical(u + 0x7FFF + lsb, 16)


def _mm_body(g_ref, we_ref, wo_ref, o_ref):
    g = g_ref[...]
    dn = (((0,), (1,)), ((), ()))
    me = lax.dot_general(we_ref[0], g, dimension_numbers=dn,
                         preferred_element_type=jnp.float32)
    mo = lax.dot_general(wo_ref[0], g, dimension_numbers=dn,
                         preferred_element_type=jnp.float32)
    be = _rte_bf16_bits(lax.bitcast_convert_type(me, jnp.int32))
    bo = _rte_bf16_bits(lax.bitcast_convert_type(mo, jnp.int32))
    o_ref[...] = jnp.bitwise_or(lax.shift_left(bo, 16), be)


def _tc_matmul_t(g, we, wo, seg):
    kmap = lambda i: (seg * KSEG + i // (E_PAD // MM_TILE), 0, 0)
    return pl.pallas_call(
        _mm_body,
        grid=(E_SEG // MM_TILE,),
        in_specs=[
            pl.BlockSpec((MM_TILE, C), lambda i: (i, 0)),
            pl.BlockSpec((1, C, C // 2), kmap),
            pl.BlockSpec((1, C, C // 2), kmap),
        ],
        out_specs=pl.BlockSpec((C // 2, MM_TILE), lambda i: (0, i)),
        out_shape=jax.ShapeDtypeStruct((C // 2, E_SEG), jnp.int32),
    )(g, we, wo)


def _red_body(x_ref, o_ref):
    i = pl.program_id(0)
    x = x_ref[...]
    col = lax.broadcasted_iota(jnp.int32, (C, RED_TILE), 1) + i * RED_TILE
    x = jnp.where(col < N, x, 0.0)
    ps = jnp.sum(x, axis=1, keepdims=True)
    pss = jnp.sum(x * x, axis=1, keepdims=True)

    @pl.when(i == 0)
    def _():
        o_ref[...] = jnp.zeros_like(o_ref)

    o_ref[:, 0:1] = o_ref[:, 0:1] + ps
    o_ref[:, 1:2] = o_ref[:, 1:2] + pss


def _tc_reduce(ot2):
    return pl.pallas_call(
        _red_body,
        grid=(N_ACC // RED_TILE,),
        in_specs=[pl.BlockSpec((C, RED_TILE), lambda i: (0, i))],
        out_specs=pl.BlockSpec((C, 128), lambda i: (0, 0)),
        out_shape=jax.ShapeDtypeStruct((C, 128), jnp.float32),
    )(ot2)


def _apply_body(x_ref, st_ref, gb_ref, o_ref):
    x = x_ref[...]
    inv_n = 1.0 / N
    mean = st_ref[:, 0:1] * inv_n
    var = st_ref[:, 1:2] * inv_n - mean * mean
    scale = gb_ref[:, 0:1] * lax.rsqrt(var + BN_EPS)
    shift = gb_ref[:, 1:2] - mean * scale
    y = jnp.maximum(x * scale + shift, 0.0)
    r = lax.broadcasted_iota(jnp.int32, (C, C), 0)
    cc = lax.broadcasted_iota(jnp.int32, (C, C), 1)
    eye = jnp.where(r == cc, 1.0, 0.0).astype(jnp.float32)
    o_ref[...] = lax.dot_general(
        y, eye, dimension_numbers=(((0,), (0,)), ((), ())),
        preferred_element_type=jnp.float32)


def _tc_apply(ot2, stats, gb):
    return pl.pallas_call(
        _apply_body,
        grid=(N_ACC // RED_TILE,),
        in_specs=[
            pl.BlockSpec((C, RED_TILE), lambda i: (0, i)),
            pl.BlockSpec((C, 128), lambda i: (0, 0)),
            pl.BlockSpec((C, 128), lambda i: (0, 0)),
        ],
        out_specs=pl.BlockSpec((RED_TILE, C), lambda i: (i, 0)),
        out_shape=jax.ShapeDtypeStruct((N, C), jnp.float32),
    )(ot2, stats, gb)


@jax.jit
def kernel(feats, edge_src, edge_dst, W, gamma, beta):
    src_p, dst_p = _sc_pad(edge_src.astype(jnp.int32).reshape(-1),
                           edge_dst.astype(jnp.int32).reshape(-1))
    src4 = src_p.reshape(NSEG * NW, CW_SEG, 128)
    dst2 = dst_p.reshape(IDX_ROWS, 128)

    we, wo = W[:, :, 0::2], W[:, :, 1::2]
    mt3s = []
    for seg in range(NSEG):
        g = _sc_gather(feats, src4, seg)
        mt = _tc_matmul_t(g, we, wo, seg)
        mt3s.append(mt.reshape(C // 2, E_SEG // 128, 128))
    ot = _sc_scatter(mt3s, dst2)
    ot2 = ot.reshape(C, N_ACC)
    stats = _tc_reduce(ot2)
    gb = jnp.zeros((C, 128), jnp.float32)
    gb = gb.at[:, 0].set(gamma).at[:, 1].set(beta)
    return _tc_apply(ot2, stats, gb)


# speedup vs baseline: 1.0940x; 1.0007x over previous
import jax
import jax.numpy as jnp
from jax import lax
from jax.experimental import pallas as pl
from jax.experimental.pallas import tpu as pltpu
from jax.experimental.pallas import tpu_sc as plsc

N = 50000
C = 128
K = 27
E = 20000
BN_EPS = 1e-5

NC, NS, L = 2, 16, 16
NW = NC * NS

E_PAD = 20480
KE_PAD = K * E_PAD
IDX_ROWS = KE_PAD // 128
EW = KE_PAD // NW

GR = 128
G_CHUNKS = EW // GR
SR = 24
S_CHUNKS = IDX_ROWS // SR

N_ACC = 51200
DUMMY_DST = N

MM_TILE = 2048
RED_TILE = 2048
RED_GRID = (N_ACC + RED_TILE - 1) // RED_TILE


PW = E // (NW - 1) // 32 * 32
TAILW = E_PAD - (NW - 1) * PW


def _pad_body(sf_hbm, df_hbm, so_hbm, do_hbm, sbuf, dbuf, s2, d2, sem):
    c = lax.axis_index("c")
    s = lax.axis_index("s")
    wid = s * NC + c

    @pl.when(wid < NW - 1)
    def _():
        w0 = wid * PW
        sb, db = (sbuf, s2), (dbuf, d2)

        def wait_w(kk):
            b = kk % 2
            pltpu.make_async_copy(sb[b], so_hbm.at[pl.ds(0, PW)],
                                  sem).wait()
            pltpu.make_async_copy(db[b], do_hbm.at[pl.ds(0, PW)],
                                  sem).wait()

        for k in range(K):
            b = k % 2
            if k >= 2:
                wait_w(k - 2)
            pltpu.sync_copy(sf_hbm.at[pl.ds(k * E + w0, PW)], sb[b])
            pltpu.sync_copy(df_hbm.at[pl.ds(k * E + w0, PW)], db[b])
            pltpu.async_copy(sb[b], so_hbm.at[pl.ds(k * E_PAD + w0, PW)],
                             sem)
            pltpu.async_copy(db[b], do_hbm.at[pl.ds(k * E_PAD + w0, PW)],
                             sem)
        wait_w(K - 2)
        wait_w(K - 1)

    @pl.when(wid == NW - 1)
    def _():
        tail = E - (NW - 1) * PW
        def fill(i, carry):
            sbuf[pl.ds(tail + i * L, L)] = jnp.zeros((L,), jnp.int32)
            dbuf[pl.ds(tail + i * L, L)] = jnp.full((L,), DUMMY_DST,
                                                    jnp.int32)
            return carry
        lax.fori_loop(0, (TAILW - tail) // L, fill, 0)
        for k in range(K):
            pltpu.sync_copy(sf_hbm.at[pl.ds(k * E + (NW - 1) * PW, tail)],
                            sbuf.at[pl.ds(0, tail)])
            pltpu.sync_copy(df_hbm.at[pl.ds(k * E + (NW - 1) * PW, tail)],
                            dbuf.at[pl.ds(0, tail)])
            pltpu.sync_copy(sbuf,
                            so_hbm.at[pl.ds(k * E_PAD + (NW - 1) * PW,
                                            TAILW)])
            pltpu.sync_copy(dbuf,
                            do_hbm.at[pl.ds(k * E_PAD + (NW - 1) * PW,
                                            TAILW)])


def _sc_pad(src_flat, dst_flat):
    mesh = plsc.VectorSubcoreMesh(core_axis_name="c", subcore_axis_name="s")
    return pl.kernel(
        _pad_body,
        out_type=(jax.ShapeDtypeStruct((KE_PAD,), jnp.int32),
                  jax.ShapeDtypeStruct((KE_PAD,), jnp.int32)),
        mesh=mesh,
        scratch_types=[
            pltpu.VMEM((TAILW,), jnp.int32),
            pltpu.VMEM((TAILW,), jnp.int32),
            pltpu.VMEM((TAILW,), jnp.int32),
            pltpu.VMEM((TAILW,), jnp.int32),
            pltpu.SemaphoreType.DMA,
        ],
        compiler_params=pltpu.CompilerParams(needs_layout_passes=False),
    )(src_flat, dst_flat)


NBUF = 5
NSEG = 3
KSEG = K // NSEG
E_SEG = KSEG * E_PAD
CW_SEG = E_SEG // NW // 128


def _make_gather_body(seg):
    def _gather_body(feats_hbm, src_hbm, g_hbm, idx_v, *bufs_and_sems):
        bufs = bufs_and_sems[:NBUF]
        gsems = bufs_and_sems[NBUF:2 * NBUF]
        wsems = bufs_and_sems[2 * NBUF:3 * NBUF]
        c = lax.axis_index("c")
        s = lax.axis_index("s")
        wid = s * NC + c
        row0 = wid * CW_SEG
        pltpu.sync_copy(src_hbm.at[seg * NW + wid], idx_v)

        def fire_g(j, b):
            pltpu.async_copy(feats_hbm.at[idx_v.at[j]], bufs[b], gsems[b])

        def wait_g(b):
            pltpu.make_async_copy(feats_hbm.at[pl.ds(0, GR)],
                                  bufs[b], gsems[b]).wait()

        def fire_w(j, b):
            off = pl.multiple_of((row0 + j) * GR, GR)
            pltpu.async_copy(bufs[b], g_hbm.at[pl.ds(off, GR)], wsems[b])

        def wait_w(b):
            pltpu.make_async_copy(bufs[b], g_hbm.at[pl.ds(0, GR)],
                                  wsems[b]).wait()

        def body(i, carry):
            for b in range(NBUF):
                @pl.when(i > 0)
                def _():
                    wait_w(b)
                fire_g(i * NBUF + b, b)
            for b in range(NBUF):
                wait_g(b)
                fire_w(i * NBUF + b, b)
            return carry

        lax.fori_loop(0, CW_SEG // NBUF, body, 0)
        for b in range(NBUF):
            wait_w(b)
    return _gather_body


def _sc_gather(feats, src4, seg):
    mesh = plsc.VectorSubcoreMesh(core_axis_name="c", subcore_axis_name="s")
    return pl.kernel(
        _make_gather_body(seg),
        out_type=jax.ShapeDtypeStruct((E_SEG, C), jnp.float32),
        mesh=mesh,
        scratch_types=[pltpu.VMEM((CW_SEG, 128), jnp.int32)]
        + [pltpu.VMEM((GR, C), jnp.float32)] * NBUF
        + [pltpu.SemaphoreType.DMA] * (2 * NBUF),
        compiler_params=pltpu.CompilerParams(needs_layout_passes=False),
        name=f"gather_seg{seg}",
    )(feats, src4)


SBUF = 3


SC_SEG = (E_SEG // 128) // SR
DCACHE_ROWS = 2848


def _scatter_body(mt0_hbm, mt1_hbm, mt2_hbm, dst_hbm, ot_hbm,
                  acc0, acc1, *bufs_and_sems):
    dbufs = bufs_and_sems[:SBUF]
    mbufs = bufs_and_sems[SBUF:2 * SBUF]
    sems = bufs_and_sems[2 * SBUF:3 * SBUF]
    c = lax.axis_index("c")
    s = lax.axis_index("s")
    wid = s * NC + c
    mts = (mt0_hbm, mt1_hbm, mt2_hbm)

    for p in range(2):
        rp = p * NW + wid

        def zero_row(r, carry):
            z = jnp.zeros((L,), jnp.float32)
            acc0[pl.ds(r * L, L)] = z
            acc1[pl.ds(r * L, L)] = z
            return carry

        lax.fori_loop(0, N_ACC // L, zero_row, 0)

        for seg in range(NSEG):
            mt_hbm = mts[seg]

            def fire(j, b):
                off = pl.multiple_of(j * SR, SR)
                pltpu.async_copy(
                    dst_hbm.at[pl.ds(seg * (E_SEG // 128) + j * SR, SR)],
                    dbufs[b], sems[b])
                pltpu.async_copy(mt_hbm.at[rp, pl.ds(off, SR)],
                                 mbufs[b], sems[b])

            def wait(b):
                pltpu.make_async_copy(dst_hbm.at[pl.ds(0, SR)],
                                      dbufs[b], sems[b]).wait()
                pltpu.make_async_copy(mt_hbm.at[0, pl.ds(0, SR)],
                                      mbufs[b], sems[b]).wait()

            def compute(b):
                dbuf, mbuf = dbufs[b], mbufs[b]

                def inner(r, carry2):
                    for v in range(8):
                        d = dbuf[r, pl.ds(v * L, L)]
                        m = mbuf[r, pl.ds(v * L, L)]
                        fe = plsc.bitcast(lax.shift_left(m, 16), jnp.float32)
                        fo = plsc.bitcast(
                            jnp.bitwise_and(m, jnp.int32(-65536)),
                            jnp.float32)
                        plsc.addupdate_scatter(acc0, [d], fe)
                        plsc.addupdate_scatter(acc1, [d], fo)
                    return carry2
                lax.fori_loop(0, SR, inner, 0)

            for b in range(SBUF):
                fire(b, b)

            def body(i, carry):
                for b in range(SBUF):
                    wait(b)
                    compute(b)

                    @pl.when(i < SC_SEG // SBUF - 1)
                    def _():
                        fire((i + 1) * SBUF + b, b)
                return carry

            lax.fori_loop(0, SC_SEG // SBUF, body, 0)
        pltpu.sync_copy(acc0, ot_hbm.at[pl.ds((2 * rp) * N_ACC, N_ACC)])
        pltpu.sync_copy(acc1, ot_hbm.at[pl.ds((2 * rp + 1) * N_ACC, N_ACC)])


def _sc_scatter(mt3s, dst2):
    mesh = plsc.VectorSubcoreMesh(core_axis_name="c", subcore_axis_name="s")
    return pl.kernel(
        _scatter_body,
        out_type=jax.ShapeDtypeStruct((C * N_ACC,), jnp.float32),
        mesh=mesh,
        scratch_types=[
            pltpu.VMEM((N_ACC,), jnp.float32),
            pltpu.VMEM((N_ACC,), jnp.float32),
        ]
        + [pltpu.VMEM((SR, 128), jnp.int32)] * SBUF
        + [pltpu.VMEM((SR, 128), jnp.int32)] * SBUF
        + [pltpu.SemaphoreType.DMA] * SBUF,
        compiler_params=pltpu.CompilerParams(needs_layout_passes=False),
    )(*mt3s, dst2)


def _rte_bf16_bits(u):
    lsb = jnp.bitwise_and(lax.shift_right_logical(u, 16), 1)
    return lax.shift_right_logical(u + 0x7FFF + lsb, 16)


def _mm_body(g_ref, we_ref, wo_ref, o_ref):
    g = g_ref[...]
    dn = (((0,), (1,)), ((), ()))
    me = lax.dot_general(we_ref[0], g, dimension_numbers=dn,
                         preferred_element_type=jnp.float32)
    mo = lax.dot_general(wo_ref[0], g, dimension_numbers=dn,
                         preferred_element_type=jnp.float32)
    be = _rte_bf16_bits(lax.bitcast_convert_type(me, jnp.int32))
    bo = _rte_bf16_bits(lax.bitcast_convert_type(mo, jnp.int32))
    o_ref[...] = jnp.bitwise_or(lax.shift_left(bo, 16), be)


def _tc_matmul_t(g, we, wo, seg):
    kmap = lambda i: (seg * KSEG + i // (E_PAD // MM_TILE), 0, 0)
    return pl.pallas_call(
        _mm_body,
        grid=(E_SEG // MM_TILE,),
        in_specs=[
            pl.BlockSpec((MM_TILE, C), lambda i: (i, 0)),
            pl.BlockSpec((1, C, C // 2), kmap),
            pl.BlockSpec((1, C, C // 2), kmap),
        ],
        out_specs=pl.BlockSpec((C // 2, MM_TILE), lambda i: (0, i)),
        out_shape=jax.ShapeDtypeStruct((C // 2, E_SEG), jnp.int32),
    )(g, we, wo)


def _red_body(x_ref, o_ref):
    i = pl.program_id(0)
    x = x_ref[...]
    col = lax.broadcasted_iota(jnp.int32, (C, RED_TILE), 1) + i * RED_TILE
    x = jnp.where(col < N, x, 0.0)
    ps = jnp.sum(x, axis=1, keepdims=True)
    pss = jnp.sum(x * x, axis=1, keepdims=True)

    @pl.when(i == 0)
    def _():
        o_ref[...] = jnp.zeros_like(o_ref)

    o_ref[:, 0:1] = o_ref[:, 0:1] + ps
    o_ref[:, 1:2] = o_ref[:, 1:2] + pss


def _tc_reduce(ot2):
    return pl.pallas_call(
        _red_body,
        grid=(RED_GRID,),
        in_specs=[pl.BlockSpec((C, RED_TILE), lambda i: (0, i))],
        out_specs=pl.BlockSpec((C, 128), lambda i: (0, 0)),
        out_shape=jax.ShapeDtypeStruct((C, 128), jnp.float32),
    )(ot2)


def _apply_body(x_ref, st_ref, gb_ref, o_ref):
    x = x_ref[...]
    inv_n = 1.0 / N
    mean = st_ref[:, 0:1] * inv_n
    var = st_ref[:, 1:2] * inv_n - mean * mean
    scale = gb_ref[:, 0:1] * lax.rsqrt(var + BN_EPS)
    shift = gb_ref[:, 1:2] - mean * scale
    y = jnp.maximum(x * scale + shift, 0.0)
    r = lax.broadcasted_iota(jnp.int32, (C, C), 0)
    cc = lax.broadcasted_iota(jnp.int32, (C, C), 1)
    eye = jnp.where(r == cc, 1.0, 0.0).astype(jnp.float32)
    o_ref[...] = lax.dot_general(
        y, eye, dimension_numbers=(((0,), (0,)), ((), ())),
        preferred_element_type=jnp.float32)


def _tc_apply(ot2, stats, gb):
    return pl.pallas_call(
        _apply_body,
        grid=(RED_GRID,),
        in_specs=[
            pl.BlockSpec((C, RED_TILE), lambda i: (0, i)),
            pl.BlockSpec((C, 128), lambda i: (0, 0)),
            pl.BlockSpec((C, 128), lambda i: (0, 0)),
        ],
        out_specs=pl.BlockSpec((RED_TILE, C), lambda i: (i, 0)),
        out_shape=jax.ShapeDtypeStruct((N, C), jnp.float32),
    )(ot2, stats, gb)


@jax.jit
def kernel(feats, edge_src, edge_dst, W, gamma, beta):
    src_p, dst_p = _sc_pad(edge_src.astype(jnp.int32).reshape(-1),
                           edge_dst.astype(jnp.int32).reshape(-1))
    src4 = src_p.reshape(NSEG * NW, CW_SEG, 128)
    dst2 = dst_p.reshape(IDX_ROWS, 128)

    we, wo = W[:, :, 0::2], W[:, :, 1::2]
    mt3s = []
    for seg in range(NSEG):
        g = _sc_gather(feats, src4, seg)
        mt = _tc_matmul_t(g, we, wo, seg)
        mt3s.append(mt.reshape(C // 2, E_SEG // 128, 128))
    ot = _sc_scatter(mt3s, dst2)
    ot2 = ot.reshape(C, N_ACC)
    stats = _tc_reduce(ot2)
    gb = jnp.zeros((C, 128), jnp.float32)
    gb = gb.at[:, 0].set(gamma).at[:, 1].set(beta)
    return _tc_apply(ot2, stats, gb)


# speedup vs baseline: 1.0941x; 1.0001x over previous
import jax
import jax.numpy as jnp
from jax import lax
from jax.experimental import pallas as pl
from jax.experimental.pallas import tpu as pltpu
from jax.experimental.pallas import tpu_sc as plsc

N = 50000
C = 128
K = 27
E = 20000
BN_EPS = 1e-5

NC, NS, L = 2, 16, 16
NW = NC * NS

E_PAD = 20480
KE_PAD = K * E_PAD
IDX_ROWS = KE_PAD // 128
EW = KE_PAD // NW

GR = 128
G_CHUNKS = EW // GR
SR = 24
S_CHUNKS = IDX_ROWS // SR

N_ACC = 51200
DUMMY_DST = N

MM_TILE = 2048
RED_TILE = 2048
RED_GRID = (N_ACC + RED_TILE - 1) // RED_TILE


PW = E // (NW - 1) // 32 * 32
TAILW = E_PAD - (NW - 1) * PW


def _pad_body(sf_hbm, df_hbm, so_hbm, do_hbm, sbuf, dbuf, s2, d2, sem):
    c = lax.axis_index("c")
    s = lax.axis_index("s")
    wid = s * NC + c

    @pl.when(wid < NW - 1)
    def _():
        w0 = wid * PW
        sb, db = (sbuf, s2), (dbuf, d2)

        def wait_w(kk):
            b = kk % 2
            pltpu.make_async_copy(sb[b], so_hbm.at[pl.ds(0, PW)],
                                  sem).wait()
            pltpu.make_async_copy(db[b], do_hbm.at[pl.ds(0, PW)],
                                  sem).wait()

        for k in range(K):
            b = k % 2
            if k >= 2:
                wait_w(k - 2)
            pltpu.sync_copy(sf_hbm.at[pl.ds(k * E + w0, PW)], sb[b])
            pltpu.sync_copy(df_hbm.at[pl.ds(k * E + w0, PW)], db[b])
            pltpu.async_copy(sb[b], so_hbm.at[pl.ds(k * E_PAD + w0, PW)],
                             sem)
            pltpu.async_copy(db[b], do_hbm.at[pl.ds(k * E_PAD + w0, PW)],
                             sem)
        wait_w(K - 2)
        wait_w(K - 1)

    @pl.when(wid == NW - 1)
    def _():
        tail = E - (NW - 1) * PW
        def fill(i, carry):
            sbuf[pl.ds(tail + i * L, L)] = jnp.zeros((L,), jnp.int32)
            dbuf[pl.ds(tail + i * L, L)] = jnp.full((L,), DUMMY_DST,
                                                    jnp.int32)
            return carry
        lax.fori_loop(0, (TAILW - tail) // L, fill, 0)
        for k in range(K):
            pltpu.sync_copy(sf_hbm.at[pl.ds(k * E + (NW - 1) * PW, tail)],
                            sbuf.at[pl.ds(0, tail)])
            pltpu.sync_copy(df_hbm.at[pl.ds(k * E + (NW - 1) * PW, tail)],
                            dbuf.at[pl.ds(0, tail)])
            pltpu.sync_copy(sbuf,
                            so_hbm.at[pl.ds(k * E_PAD + (NW - 1) * PW,
                                            TAILW)])
            pltpu.sync_copy(dbuf,
                            do_hbm.at[pl.ds(k * E_PAD + (NW - 1) * PW,
                                            TAILW)])


def _sc_pad(src_flat, dst_flat):
    mesh = plsc.VectorSubcoreMesh(core_axis_name="c", subcore_axis_name="s")
    return pl.kernel(
        _pad_body,
        out_type=(jax.ShapeDtypeStruct((KE_PAD,), jnp.int32),
                  jax.ShapeDtypeStruct((KE_PAD,), jnp.int32)),
        mesh=mesh,
        scratch_types=[
            pltpu.VMEM((TAILW,), jnp.int32),
            pltpu.VMEM((TAILW,), jnp.int32),
            pltpu.VMEM((TAILW,), jnp.int32),
            pltpu.VMEM((TAILW,), jnp.int32),
            pltpu.SemaphoreType.DMA,
        ],
        compiler_params=pltpu.CompilerParams(needs_layout_passes=False),
    )(src_flat, dst_flat)


NBUF = 5
NSEG = 3
KSEG = K // NSEG
E_SEG = KSEG * E_PAD
CW_SEG = E_SEG // NW // 128


def _make_gather_body(seg):
    def _gather_body(feats_hbm, src_hbm, g_hbm, idx_v, *bufs_and_sems):
        bufs = bufs_and_sems[:NBUF]
        gsems = bufs_and_sems[NBUF:2 * NBUF]
        wsems = bufs_and_sems[2 * NBUF:3 * NBUF]
        c = lax.axis_index("c")
        s = lax.axis_index("s")
        wid = s * NC + c
        row0 = wid * CW_SEG
        pltpu.sync_copy(src_hbm.at[seg * NW + wid], idx_v)

        def fire_g(j, b):
            pltpu.async_copy(feats_hbm.at[idx_v.at[j]], bufs[b], gsems[b])

        def wait_g(b):
            pltpu.make_async_copy(feats_hbm.at[pl.ds(0, GR)],
                                  bufs[b], gsems[b]).wait()

        def fire_w(j, b):
            off = pl.multiple_of((row0 + j) * GR, GR)
            pltpu.async_copy(bufs[b], g_hbm.at[pl.ds(off, GR)], wsems[b])

        def wait_w(b):
            pltpu.make_async_copy(bufs[b], g_hbm.at[pl.ds(0, GR)],
                                  wsems[b]).wait()

        def body(i, carry):
            for b in range(NBUF):
                @pl.when(i > 0)
                def _():
                    wait_w(b)
                fire_g(i * NBUF + b, b)
            for b in range(NBUF):
                wait_g(b)
                fire_w(i * NBUF + b, b)
            return carry

        lax.fori_loop(0, CW_SEG // NBUF, body, 0)
        for b in range(NBUF):
            wait_w(b)
    return _gather_body


def _sc_gather(feats, src4, seg):
    mesh = plsc.VectorSubcoreMesh(core_axis_name="c", subcore_axis_name="s")
    return pl.kernel(
        _make_gather_body(seg),
        out_type=jax.ShapeDtypeStruct((E_SEG, C), jnp.float32),
        mesh=mesh,
        scratch_types=[pltpu.VMEM((CW_SEG, 128), jnp.int32)]
        + [pltpu.VMEM((GR, C), jnp.float32)] * NBUF
        + [pltpu.SemaphoreType.DMA] * (2 * NBUF),
        compiler_params=pltpu.CompilerParams(needs_layout_passes=False),
        name=f"gather_seg{seg}",
    )(feats, src4)


SBUF = 3


SC_SEG = (E_SEG // 128) // SR


def _scatter_body(mt0_hbm, mt1_hbm, mt2_hbm, dst_hbm, ot_hbm,
                  acc0, acc1, *bufs_and_sems):
    dbufs = bufs_and_sems[:SBUF]
    mbufs = bufs_and_sems[SBUF:2 * SBUF]
    sems = bufs_and_sems[2 * SBUF:3 * SBUF]
    c = lax.axis_index("c")
    s = lax.axis_index("s")
    wid = s * NC + c
    mts = (mt0_hbm, mt1_hbm, mt2_hbm)

    for p in range(2):
        rp = p * NW + wid

        def zero_row(r, carry):
            z = jnp.zeros((L,), jnp.float32)
            acc0[pl.ds(r * L, L)] = z
            acc1[pl.ds(r * L, L)] = z
            return carry

        lax.fori_loop(0, N_ACC // L, zero_row, 0)

        for seg in range(NSEG):
            mt_hbm = mts[seg]

            def fire(j, b):
                off = pl.multiple_of(j * SR, SR)
                pltpu.async_copy(
                    dst_hbm.at[pl.ds(seg * (E_SEG // 128) + j * SR, SR)],
                    dbufs[b], sems[b])
                pltpu.async_copy(mt_hbm.at[rp, pl.ds(off, SR)],
                                 mbufs[b], sems[b])

            def wait(b):
                pltpu.make_async_copy(dst_hbm.at[pl.ds(0, SR)],
                                      dbufs[b], sems[b]).wait()
                pltpu.make_async_copy(mt_hbm.at[0, pl.ds(0, SR)],
                                      mbufs[b], sems[b]).wait()

            def compute(b):
                dbuf, mbuf = dbufs[b], mbufs[b]

                def inner(r, carry2):
                    for v in range(8):
                        d = dbuf[r, pl.ds(v * L, L)]
                        m = mbuf[r, pl.ds(v * L, L)]
                        fe = plsc.bitcast(lax.shift_left(m, 16), jnp.float32)
                        fo = plsc.bitcast(
                            jnp.bitwise_and(m, jnp.int32(-65536)),
                            jnp.float32)
                        plsc.addupdate_scatter(acc0, [d], fe)
                        plsc.addupdate_scatter(acc1, [d], fo)
                    return carry2
                lax.fori_loop(0, SR, inner, 0)

            for b in range(SBUF):
                fire(b, b)

            def body(i, carry):
                for b in range(SBUF):
                    wait(b)
                    compute(b)

                    @pl.when(i < SC_SEG // SBUF - 1)
                    def _():
                        fire((i + 1) * SBUF + b, b)
                return carry

            lax.fori_loop(0, SC_SEG // SBUF, body, 0)
        pltpu.sync_copy(acc0, ot_hbm.at[pl.ds((2 * rp) * N_ACC, N_ACC)])
        pltpu.sync_copy(acc1, ot_hbm.at[pl.ds((2 * rp + 1) * N_ACC, N_ACC)])


def _sc_scatter(mt3s, dst2):
    mesh = plsc.VectorSubcoreMesh(core_axis_name="c", subcore_axis_name="s")
    return pl.kernel(
        _scatter_body,
        out_type=jax.ShapeDtypeStruct((C * N_ACC,), jnp.float32),
        mesh=mesh,
        scratch_types=[
            pltpu.VMEM((N_ACC,), jnp.float32),
            pltpu.VMEM((N_ACC,), jnp.float32),
        ]
        + [pltpu.VMEM((SR, 128), jnp.int32)] * SBUF
        + [pltpu.VMEM((SR, 128), jnp.int32)] * SBUF
        + [pltpu.SemaphoreType.DMA] * SBUF,
        compiler_params=pltpu.CompilerParams(needs_layout_passes=False),
    )(*mt3s, dst2)


def _rte_bf16_bits(u):
    lsb = jnp.bitwise_and(lax.shift_right_logical(u, 16), 1)
    return lax.shift_right_logical(u + 0x7FFF + lsb, 16)


def _mm_body(g_ref, we_ref, wo_ref, o_ref):
    g = g_ref[...]
    dn = (((0,), (1,)), ((), ()))
    me = lax.dot_general(we_ref[0], g, dimension_numbers=dn,
                         preferred_element_type=jnp.float32)
    mo = lax.dot_general(wo_ref[0], g, dimension_numbers=dn,
                         preferred_element_type=jnp.float32)
    be = _rte_bf16_bits(lax.bitcast_convert_type(me, jnp.int32))
    bo = _rte_bf16_bits(lax.bitcast_convert_type(mo, jnp.int32))
    o_ref[...] = jnp.bitwise_or(lax.shift_left(bo, 16), be)


def _tc_matmul_t(g, we, wo, seg):
    kmap = lambda i: (seg * KSEG + i // (E_PAD // MM_TILE), 0, 0)
    return pl.pallas_call(
        _mm_body,
        grid=(E_SEG // MM_TILE,),
        in_specs=[
            pl.BlockSpec((MM_TILE, C), lambda i: (i, 0)),
            pl.BlockSpec((1, C, C // 2), kmap),
            pl.BlockSpec((1, C, C // 2), kmap),
        ],
        out_specs=pl.BlockSpec((C // 2, MM_TILE), lambda i: (0, i)),
        out_shape=jax.ShapeDtypeStruct((C // 2, E_SEG), jnp.int32),
    )(g, we, wo)


def _red_body(x_ref, o_ref):
    i = pl.program_id(0)
    x = x_ref[...]
    col = lax.broadcasted_iota(jnp.int32, (C, RED_TILE), 1) + i * RED_TILE
    x = jnp.where(col < N, x, 0.0)
    ps = jnp.sum(x, axis=1, keepdims=True)
    pss = jnp.sum(x * x, axis=1, keepdims=True)

    @pl.when(i == 0)
    def _():
        o_ref[...] = jnp.zeros_like(o_ref)

    o_ref[:, 0:1] = o_ref[:, 0:1] + ps
    o_ref[:, 1:2] = o_ref[:, 1:2] + pss


def _tc_reduce(ot2):
    return pl.pallas_call(
        _red_body,
        grid=(RED_GRID,),
        in_specs=[pl.BlockSpec((C, RED_TILE), lambda i: (0, i))],
        out_specs=pl.BlockSpec((C, 128), lambda i: (0, 0)),
        out_shape=jax.ShapeDtypeStruct((C, 128), jnp.float32),
    )(ot2)


def _apply_body(x_ref, st_ref, gb_ref, o_ref):
    x = x_ref[...]
    inv_n = 1.0 / N
    mean = st_ref[:, 0:1] * inv_n
    var = st_ref[:, 1:2] * inv_n - mean * mean
    scale = gb_ref[:, 0:1] * lax.rsqrt(var + BN_EPS)
    shift = gb_ref[:, 1:2] - mean * scale
    y = jnp.maximum(x * scale + shift, 0.0)
    r = lax.broadcasted_iota(jnp.int32, (C, C), 0)
    cc = lax.broadcasted_iota(jnp.int32, (C, C), 1)
    eye = jnp.where(r == cc, 1.0, 0.0).astype(jnp.float32)
    o_ref[...] = lax.dot_general(
        y, eye, dimension_numbers=(((0,), (0,)), ((), ())),
        preferred_element_type=jnp.float32)


def _tc_apply(ot2, stats, gb):
    return pl.pallas_call(
        _apply_body,
        grid=(RED_GRID,),
        in_specs=[
            pl.BlockSpec((C, RED_TILE), lambda i: (0, i)),
            pl.BlockSpec((C, 128), lambda i: (0, 0)),
            pl.BlockSpec((C, 128), lambda i: (0, 0)),
        ],
        out_specs=pl.BlockSpec((RED_TILE, C), lambda i: (i, 0)),
        out_shape=jax.ShapeDtypeStruct((N, C), jnp.float32),
    )(ot2, stats, gb)


@jax.jit
def kernel(feats, edge_src, edge_dst, W, gamma, beta):
    src_p, dst_p = _sc_pad(edge_src.astype(jnp.int32).reshape(-1),
                           edge_dst.astype(jnp.int32).reshape(-1))
    src4 = src_p.reshape(NSEG * NW, CW_SEG, 128)
    dst2 = dst_p.reshape(IDX_ROWS, 128)

    we, wo = W[:, :, 0::2], W[:, :, 1::2]
    mt3s = []
    for seg in range(NSEG):
        g = _sc_gather(feats, src4, seg)
        mt = _tc_matmul_t(g, we, wo, seg)
        mt3s.append(mt.reshape(C // 2, E_SEG // 128, 128))
    ot = _sc_scatter(mt3s, dst2)
    ot2 = ot.reshape(C, N_ACC)
    stats = _tc_reduce(ot2)
    gb = jnp.zeros((C, 128), jnp.float32)
    gb = gb.at[:, 0].set(gamma).at[:, 1].set(beta)
    return _tc_apply(ot2, stats, gb)
